# baseline (speedup 1.0000x reference)
import functools
import math

import jax
import jax.numpy as jnp
from jax import lax
from jax.experimental import pallas as pl
from jax.experimental.pallas import tpu as pltpu
from jax.experimental.pallas import tpu_sc as plsc

IN = 256
HID = 256
H = 4
DH = 64
B = 128
N = 50000
BN = 1024
NBLK = (N + BN - 1) // BN
NPAD = NBLK * BN
SHIFTS = [1 << k for k in range((BN - 1).bit_length())]

NC = 2
NS = 16
NW = NC * NS
RPT = NPAD // NW
SC_CHUNK = 112
SC_NCHUNK = RPT // SC_CHUNK
SC_UNROLL = 4
assert SC_CHUNK * SC_NCHUNK == RPT and SC_CHUNK % 8 == 0
assert SC_CHUNK % SC_UNROLL == 0

_NEG = float('-inf')


def _roll(a, s, axis=0):
    s = s % a.shape[axis]
    if s == 0:
        return a
    if axis == 0:
        return jnp.concatenate([a[-s:], a[:-s]], axis=0)
    return jnp.concatenate([a[:, -s:], a[:, :-s]], axis=1)


def _expand_heads(e):
    return jnp.concatenate(
        [jnp.broadcast_to(e[:, h:h + 1], (e.shape[0], DH)) for h in range(H)],
        axis=1)


def _head_sel():
    col = lax.broadcasted_iota(jnp.int32, (H, HID), 1)
    row = lax.broadcasted_iota(jnp.int32, (H, HID), 0)
    return (col // DH == row).astype(jnp.float32)


def _dot(a, b, dims):
    return lax.dot_general(a, b, (dims, ((), ())),
                           preferred_element_type=jnp.float32)


def _sc_maxpool_body(x_hbm, ids_hbm, out_hbm,
                     xv0, xv1, id0, id1, acc, sx0, sx1, si0, si1):
    wid = lax.axis_index("s") * NC + lax.axis_index("c")
    base = wid * RPT
    NV = IN // 16

    def init_row(b, carry):
        for k in range(NV):
            acc[pl.ds(b * IN + 16 * k, 16)] = jnp.full((16,), _NEG,
                                                       jnp.float32)
        return carry
    lax.fori_loop(0, B + 1, init_row, 0)

    def start(c, xv, idv, sx, si):
        c = jnp.minimum(c, SC_NCHUNK - 1)
        pltpu.async_copy(
            x_hbm.at[pl.ds((base + c * SC_CHUNK) * IN, SC_CHUNK * IN)],
            xv, sx)
        pltpu.async_copy(ids_hbm.at[pl.ds(base + c * SC_CHUNK, SC_CHUNK)],
                         idv.at[pl.ds(0, SC_CHUNK)], si)

    def wait(xv, idv, sx, si):
        pltpu.make_async_copy(x_hbm.at[pl.ds(0, SC_CHUNK * IN)],
                              xv, sx).wait()
        pltpu.make_async_copy(ids_hbm.at[pl.ds(0, SC_CHUNK)],
                              idv.at[pl.ds(0, SC_CHUNK)], si).wait()

    def process(xv, idv, carry):
        def do_rows(jj, carry):
            for u in range(SC_UNROLL):
                j = jj * SC_UNROLL + u
                prev = carry[0]
                sid = idv[pl.ds(j, 16)][0]
                fresh = sid != prev

                @pl.when(fresh & (prev >= 0))
                def _flush(carry=carry, prev=prev):
                    for k in range(NV):
                        acc[pl.ds(prev * IN + 16 * k, 16)] = carry[1 + k]

                new_m = []
                for k in range(NV):
                    xvec = xv[pl.ds(j * IN + 16 * k, 16)]
                    new_m.append(jnp.where(fresh, xvec,
                                           jnp.maximum(carry[1 + k], xvec)))
                carry = (sid, *new_m)
            return carry

        return lax.fori_loop(0, SC_CHUNK // SC_UNROLL, do_rows, carry)

    zeros = [jnp.zeros((16,), jnp.float32) for _ in range(IN // 16)]
    start(0, xv0, id0, sx0, si0)

    def do_pair(cc, carry):
        c0 = 2 * cc
        start(c0 + 1, xv1, id1, sx1, si1)
        wait(xv0, id0, sx0, si0)
        carry = process(xv0, id0, carry)
        start(c0 + 2, xv0, id0, sx0, si0)
        wait(xv1, id1, sx1, si1)
        carry = process(xv1, id1, carry)
        return carry

    carry = lax.fori_loop(0, SC_NCHUNK // 2, do_pair,
                          (jnp.int32(-1), *zeros))
    wait(xv0, id0, sx0, si0)
    last = carry[0]

    @pl.when(last >= 0)
    def _final_flush():
        for k in range(NV):
            acc[pl.ds(last * IN + 16 * k, 16)] = carry[1 + k]

    pltpu.sync_copy(acc.at[pl.ds(0, B * IN)],
                    out_hbm.at[pl.ds(wid * B * IN, B * IN)])


def _sc_maxpool(xp, idflat):
    mesh = plsc.VectorSubcoreMesh(core_axis_name="c", subcore_axis_name="s")
    f = pl.kernel(
        _sc_maxpool_body,
        mesh=mesh,
        out_type=jax.ShapeDtypeStruct((NW * B * IN,), jnp.float32),
        scratch_types=[
            pltpu.VMEM((SC_CHUNK * IN,), jnp.float32),
            pltpu.VMEM((SC_CHUNK * IN,), jnp.float32),
            pltpu.VMEM((SC_CHUNK + 16,), jnp.int32),
            pltpu.VMEM((SC_CHUNK + 16,), jnp.int32),
            pltpu.VMEM(((B + 1) * IN,), jnp.float32),
            pltpu.SemaphoreType.DMA,
            pltpu.SemaphoreType.DMA,
            pltpu.SemaphoreType.DMA,
            pltpu.SemaphoreType.DMA,
        ],
    )
    return f(xp.reshape(-1), idflat).reshape(NW, B, IN)


def _main_body(x_ref, idsr_ref, wk_ref, q_ref, bk_ref, wv_ref,
               counts_ref, sumx_ref, m_ref, d_ref, segvv_ref,
               wq_s, bq_s):
    i = pl.program_id(0)

    @pl.when(i == 0)
    def _init():
        counts_ref[...] = jnp.zeros_like(counts_ref)
        sumx_ref[...] = jnp.zeros_like(sumx_ref)
        m_ref[...] = jnp.full_like(m_ref, _NEG)
        d_ref[...] = jnp.zeros_like(d_ref)
        segvv_ref[...] = jnp.zeros_like(segvv_ref)
        q = q_ref[...]
        col = lax.broadcasted_iota(jnp.int32, (H, IN), 1)
        row = lax.broadcasted_iota(jnp.int32, (H, IN), 0)
        qtile = jnp.concatenate([q] * (IN // DH), axis=1)
        qblk = jnp.where(col // DH == row, qtile, 0.0)
        wq_s[...] = _dot(qblk, wk_ref[...], ((1,), (0,)))
        bq_s[...] = _dot(qblk, bk_ref[...], ((1,), (0,)))

    xb = x_ref[...]
    ids_row = idsr_ref[0]
    scores = _dot(wq_s[...], xb, ((1,), (1,))) + bq_s[...]

    segc = lax.broadcasted_iota(jnp.int32, (B, BN), 0)
    Pt = (ids_row == segc).astype(jnp.float32)

    cnt8 = _dot(Pt, jnp.ones((BN, 8), jnp.float32), ((1,), (0,)))
    counts_ref[...] += cnt8
    present = cnt8[:, :1] > 0.0
    sumx_ref[...] += _dot(Pt, xb, ((1,), (0,)))

    m_s = scores
    for s in SHIFTS:
        same = ids_row == _roll(ids_row, s, 1)
        m_s = jnp.maximum(m_s, jnp.where(same, _roll(m_s, s, 1), _NEG))
    lane = lax.broadcasted_iota(jnp.int32, (1, BN), 1)
    lastrow = (ids_row != _roll(ids_row, -1, 1)) | (lane == BN - 1)
    Eft = Pt * lastrow.astype(jnp.float32)
    blkmax_s = _dot(Eft, m_s, ((1,), (1,)))

    m_old = m_ref[...]
    m_new = jnp.maximum(m_old, jnp.where(present, blkmax_s, _NEG))
    r = jnp.where(jnp.isfinite(m_old), jnp.exp(m_old - m_new), 0.0)
    m_g = jnp.where(jnp.isfinite(m_new), m_new, 0.0)
    smax_at = _dot(m_g, Pt, ((0,), (0,)))
    ex = jnp.exp(scores - smax_at)
    d_ref[...] = d_ref[...] * r + _dot(Pt, ex, ((1,), (1,)))

    sel = _head_sel()
    vals = _dot(xb, wv_ref[...], ((1,), (1,)))
    weighted = vals * _dot(ex, sel, ((0,), (0,)))
    r_exp = _dot(r, sel, ((1,), (0,)))
    segvv_ref[...] = segvv_ref[...] * r_exp + _dot(Pt, weighted, ((1,), (0,)))
    m_ref[...] = m_new


def _ln(x, g, b, eps=1e-5):
    mu = jnp.mean(x, axis=-1, keepdims=True)
    var = jnp.mean((x - mu) ** 2, axis=-1, keepdims=True)
    return (x - mu) * lax.rsqrt(var + eps) * g + b


def _final_body(counts_ref, sumx_ref, mparts_ref, d_ref, segvv_ref,
                bv_ref, wo_ref, bo_ref, wg_ref, bg_ref, l1g_ref, l1b_ref,
                wm_ref, bm_ref, wx_ref, bx_ref, ws_ref, bs_ref,
                wgm_ref, bgm_ref, wgx_ref, bgx_ref, wgs_ref, bgs_ref,
                wp_ref, bp_ref, l2g_ref, l2b_ref,
                wf1_ref, bf1_ref, lfg_ref, lfb_ref, wf2_ref, bf2_ref,
                out_ref):
    counts = counts_ref[:, :1]
    nonempty = counts > 0.0
    sum_x = sumx_ref[...]
    avg = sum_x / jnp.maximum(counts, 1.0)

    maxp = mparts_ref[0]
    for t in range(1, NW):
        maxp = jnp.maximum(maxp, mparts_ref[t])
    max_pool = jnp.where(jnp.isfinite(maxp), maxp, 0.0)

    d = d_ref[...]
    d_safe = jnp.where(d == 0.0, 1.0, d)
    wv = segvv_ref[...] / _expand_heads(d_safe)
    wv = wv + bv_ref[...] * nonempty.astype(jnp.float32)
    ctx = _dot(wv, wo_ref[...], ((1,), (1,))) + bo_ref[...]

    comb = jnp.concatenate([ctx, avg], axis=1)
    gate = jax.nn.sigmoid(_dot(comb, wg_ref[...], ((1,), (1,))) + bg_ref[...])
    ctx = gate * ctx + (1.0 - gate) * avg
    ctx = jnp.where(nonempty, ctx, 0.0)
    attn_emb = _ln(ctx, l1g_ref[...], l1b_ref[...])

    mean_r = _dot(avg, wm_ref[...], ((1,), (1,))) + bm_ref[...]
    max_r = _dot(max_pool, wx_ref[...], ((1,), (1,))) + bx_ref[...]
    sum_r = _dot(sum_x, ws_ref[...], ((1,), (1,))) + bs_ref[...]
    gm = jax.nn.sigmoid(
        jnp.sum(mean_r * wgm_ref[...], axis=1, keepdims=True) + bgm_ref[0, 0])
    gx = jax.nn.sigmoid(
        jnp.sum(max_r * wgx_ref[...], axis=1, keepdims=True) + bgx_ref[0, 0])
    gs = jax.nn.sigmoid(
        jnp.sum(sum_r * wgs_ref[...], axis=1, keepdims=True) + bgs_ref[0, 0])
    g0 = jnp.maximum(jnp.maximum(gm, gx), gs)
    em = jnp.exp(gm - g0)
    ex_ = jnp.exp(gx - g0)
    es = jnp.exp(gs - g0)
    z = em + ex_ + es
    pooled = (em * mean_r + ex_ * max_r + es * sum_r) / z
    pool_emb = _ln(_dot(pooled, wp_ref[...], ((1,), (1,))) + bp_ref[...],
                   l2g_ref[...], l2b_ref[...])

    comb2 = jnp.concatenate([attn_emb, pool_emb], axis=1)
    h1 = _dot(comb2, wf1_ref[...], ((1,), (1,))) + bf1_ref[...]
    h1 = _ln(h1, lfg_ref[...], lfb_ref[...])
    h1 = 0.5 * h1 * (1.0 + lax.erf(h1 * (1.0 / math.sqrt(2.0))))
    out_ref[...] = _dot(h1, wf2_ref[...], ((1,), (1,))) + bf2_ref[...]


def _row(v):
    return v.reshape(1, -1)


@jax.jit
def kernel(x, params, batch):
    p = params
    xp = jnp.pad(x, ((0, NPAD - N), (0, 0)))
    idflat = jnp.pad(batch, (0, NPAD - N), constant_values=B)
    idr = idflat.reshape(NBLK, 1, BN)

    maxp_parts = _sc_maxpool(xp, idflat)

    f32 = jnp.float32
    block = lambda shape: pl.BlockSpec(shape, lambda i: (0,) * len(shape))
    stats = pl.pallas_call(
        _main_body,
        grid=(NBLK,),
        in_specs=[
            pl.BlockSpec((BN, IN), lambda i: (i, 0)),
            pl.BlockSpec((1, 1, BN), lambda i: (i, 0, 0)),
            block((HID, IN)), block((H, DH)), block((HID, 1)),
            block((HID, IN)),
        ],
        out_specs=[
            block((B, 8)), block((B, IN)),
            block((B, H)), block((B, H)), block((B, HID)),
        ],
        out_shape=[
            jax.ShapeDtypeStruct((B, 8), f32),
            jax.ShapeDtypeStruct((B, IN), f32),
            jax.ShapeDtypeStruct((B, H), f32),
            jax.ShapeDtypeStruct((B, H), f32),
            jax.ShapeDtypeStruct((B, HID), f32),
        ],
        scratch_shapes=[
            pltpu.VMEM((H, IN), f32),
            pltpu.VMEM((H, 1), f32),
        ],
    )(xp, idr, p['Wk'], p['query'], p['bk'].reshape(HID, 1), p['Wv'])
    counts, sum_x, m_run, d_run, segvv = stats

    out = pl.pallas_call(
        _final_body,
        out_shape=jax.ShapeDtypeStruct((B, IN), f32),
    )(counts, sum_x, maxp_parts, d_run, segvv,
      _row(p['bv']), p['Wo'], _row(p['bo']), p['Wg'], _row(p['bg']),
      _row(p['ln1_g']), _row(p['ln1_b']),
      p['Wm'], _row(p['bm']), p['Wx'], _row(p['bx']), p['Ws'], _row(p['bs']),
      p['Wgm'], _row(p['bgm']), p['Wgx'], _row(p['bgx']),
      p['Wgs'], _row(p['bgs']),
      p['Wp'], _row(p['bp']), _row(p['ln2_g']), _row(p['ln2_b']),
      p['Wf1'], _row(p['bf1']), _row(p['lnf_g']), _row(p['lnf_b']),
      p['Wf2'], _row(p['bf2']))
    return out


# speedup vs baseline: 1.3693x; 1.3693x over previous
import functools
import math

import jax
import jax.numpy as jnp
from jax import lax
from jax.experimental import pallas as pl
from jax.experimental.pallas import tpu as pltpu
from jax.experimental.pallas import tpu_sc as plsc

IN = 256
HID = 256
H = 4
DH = 64
B = 128
N = 50000
BN = 1000
NBLK = N // BN
assert NBLK * BN == N
SHIFTS = [1 << k for k in range((BN - 1).bit_length())]

NC = 2
NS = 16
NW = NC * NS
RPT = 1560
SC_CHUNK = 120
SC_NCHUNK = RPT // SC_CHUNK
SC_UNROLL = 4
SC_TAIL = N - NW * RPT
assert SC_CHUNK * SC_NCHUNK == RPT and SC_CHUNK % 8 == 0
assert SC_CHUNK % SC_UNROLL == 0 and SC_TAIL % SC_UNROLL == 0
assert (NW * RPT) % 8 == 0 and SC_TAIL >= 0 and SC_TAIL <= SC_CHUNK

_NEG = float('-inf')


def _roll(a, s, axis=0):
    s = s % a.shape[axis]
    if s == 0:
        return a
    if axis == 0:
        return jnp.concatenate([a[-s:], a[:-s]], axis=0)
    return jnp.concatenate([a[:, -s:], a[:, :-s]], axis=1)


def _expand_heads(e):
    return jnp.concatenate(
        [jnp.broadcast_to(e[:, h:h + 1], (e.shape[0], DH)) for h in range(H)],
        axis=1)


def _head_sel():
    col = lax.broadcasted_iota(jnp.int32, (H, HID), 1)
    row = lax.broadcasted_iota(jnp.int32, (H, HID), 0)
    return (col // DH == row).astype(jnp.float32)


def _dot(a, b, dims):
    return lax.dot_general(a, b, (dims, ((), ())),
                           preferred_element_type=jnp.float32)


def _sc_maxpool_body(x_hbm, ids_hbm, out_hbm,
                     xv0, xv1, id0, id1, acc, sx0, sx1, si0, si1):
    wid = lax.axis_index("s") * NC + lax.axis_index("c")
    base = wid * RPT
    NV = IN // 16

    def init_row(b, carry):
        for k in range(NV):
            acc[pl.ds(b * IN + 16 * k, 16)] = jnp.full((16,), _NEG,
                                                       jnp.float32)
        return carry
    lax.fori_loop(0, B + 1, init_row, 0)

    def start(c, xv, idv, sx, si, nrows=SC_CHUNK, pos=None):
        off = base + c * SC_CHUNK if pos is None else pos
        pltpu.async_copy(
            x_hbm.at[pl.ds(off * IN, nrows * IN)],
            xv.at[pl.ds(0, nrows * IN)], sx)
        pltpu.async_copy(ids_hbm.at[pl.ds(off, nrows)],
                         idv.at[pl.ds(0, nrows)], si)

    def wait(xv, idv, sx, si, nrows=SC_CHUNK):
        pltpu.make_async_copy(x_hbm.at[pl.ds(0, nrows * IN)],
                              xv.at[pl.ds(0, nrows * IN)], sx).wait()
        pltpu.make_async_copy(ids_hbm.at[pl.ds(0, nrows)],
                              idv.at[pl.ds(0, nrows)], si).wait()

    def process(xv, idv, carry, nrows=SC_CHUNK):
        def do_rows(jj, carry):
            for u in range(SC_UNROLL):
                j = jj * SC_UNROLL + u
                prev = carry[0]
                sid = idv[pl.ds(j, 16)][0]
                fresh = sid != prev

                @pl.when(fresh & (prev >= 0))
                def _flush(carry=carry, prev=prev):
                    for k in range(NV):
                        acc[pl.ds(prev * IN + 16 * k, 16)] = carry[1 + k]

                new_m = []
                for k in range(NV):
                    xvec = xv[pl.ds(j * IN + 16 * k, 16)]
                    new_m.append(jnp.where(fresh, xvec,
                                           jnp.maximum(carry[1 + k], xvec)))
                carry = (sid, *new_m)
            return carry

        return lax.fori_loop(0, nrows // SC_UNROLL, do_rows, carry)

    zeros = [jnp.zeros((16,), jnp.float32) for _ in range(IN // 16)]
    start(0, xv0, id0, sx0, si0)

    def do_pair(cc, carry):
        c0 = 2 * cc
        start(c0 + 1, xv1, id1, sx1, si1)
        wait(xv0, id0, sx0, si0)
        carry = process(xv0, id0, carry)
        start(c0 + 2, xv0, id0, sx0, si0)
        wait(xv1, id1, sx1, si1)
        carry = process(xv1, id1, carry)
        return carry

    carry = lax.fori_loop(0, SC_NCHUNK // 2, do_pair,
                          (jnp.int32(-1), *zeros))
    start(0, xv1, id1, sx1, si1, nrows=SC_TAIL, pos=NW * RPT)
    wait(xv0, id0, sx0, si0)
    carry = process(xv0, id0, carry)
    wait(xv1, id1, sx1, si1, nrows=SC_TAIL)
    carry = process(xv1, id1, carry, nrows=SC_TAIL)
    last = carry[0]

    @pl.when(last >= 0)
    def _final_flush():
        for k in range(NV):
            acc[pl.ds(last * IN + 16 * k, 16)] = carry[1 + k]

    pltpu.sync_copy(acc.at[pl.ds(0, B * IN)],
                    out_hbm.at[pl.ds(wid * B * IN, B * IN)])


def _sc_maxpool(xp, idflat):
    mesh = plsc.VectorSubcoreMesh(core_axis_name="c", subcore_axis_name="s")
    f = pl.kernel(
        _sc_maxpool_body,
        mesh=mesh,
        out_type=jax.ShapeDtypeStruct((NW * B * IN,), jnp.float32),
        scratch_types=[
            pltpu.VMEM((SC_CHUNK * IN,), jnp.float32),
            pltpu.VMEM((SC_CHUNK * IN,), jnp.float32),
            pltpu.VMEM((SC_CHUNK + 16,), jnp.int32),
            pltpu.VMEM((SC_CHUNK + 16,), jnp.int32),
            pltpu.VMEM(((B + 1) * IN,), jnp.float32),
            pltpu.SemaphoreType.DMA,
            pltpu.SemaphoreType.DMA,
            pltpu.SemaphoreType.DMA,
            pltpu.SemaphoreType.DMA,
        ],
    )
    return f(xp.reshape(-1), idflat).reshape(NW, B, IN)


def _main_body(x_ref, idsr_ref, wk_ref, q_ref, bk_ref, wv_ref,
               counts_ref, sumx_ref, m_ref, d_ref, segvv_ref,
               wq_s, bq_s):
    i = pl.program_id(0)

    @pl.when(i == 0)
    def _init():
        counts_ref[...] = jnp.zeros_like(counts_ref)
        sumx_ref[...] = jnp.zeros_like(sumx_ref)
        m_ref[...] = jnp.full_like(m_ref, _NEG)
        d_ref[...] = jnp.zeros_like(d_ref)
        segvv_ref[...] = jnp.zeros_like(segvv_ref)
        q = q_ref[...]
        col = lax.broadcasted_iota(jnp.int32, (H, IN), 1)
        row = lax.broadcasted_iota(jnp.int32, (H, IN), 0)
        qtile = jnp.concatenate([q] * (IN // DH), axis=1)
        qblk = jnp.where(col // DH == row, qtile, 0.0)
        wq_s[...] = _dot(qblk, wk_ref[...], ((1,), (0,)))
        bq_s[...] = _dot(qblk, bk_ref[...], ((1,), (0,)))

    xb = x_ref[...]
    ids_row = idsr_ref[0]
    scores = _dot(wq_s[...], xb, ((1,), (1,))) + bq_s[...]

    segc = lax.broadcasted_iota(jnp.int32, (B, BN), 0)
    Pt = (ids_row == segc).astype(jnp.float32)

    cnt8 = _dot(Pt, jnp.ones((BN, 8), jnp.float32), ((1,), (0,)))
    counts_ref[...] += cnt8
    present = cnt8[:, :1] > 0.0
    sumx_ref[...] += _dot(Pt, xb, ((1,), (0,)))

    m_s = scores
    for s in SHIFTS:
        same = ids_row == _roll(ids_row, s, 1)
        m_s = jnp.maximum(m_s, jnp.where(same, _roll(m_s, s, 1), _NEG))
    lane = lax.broadcasted_iota(jnp.int32, (1, BN), 1)
    lastrow = (ids_row != _roll(ids_row, -1, 1)) | (lane == BN - 1)
    Eft = Pt * lastrow.astype(jnp.float32)
    blkmax_s = _dot(Eft, m_s, ((1,), (1,)))

    m_old = m_ref[...]
    m_new = jnp.maximum(m_old, jnp.where(present, blkmax_s, _NEG))
    r = jnp.where(jnp.isfinite(m_old), jnp.exp(m_old - m_new), 0.0)
    m_g = jnp.where(jnp.isfinite(m_new), m_new, 0.0)
    smax_at = _dot(m_g, Pt, ((0,), (0,)))
    ex = jnp.exp(scores - smax_at)
    d_ref[...] = d_ref[...] * r + _dot(Pt, ex, ((1,), (1,)))

    sel = _head_sel()
    vals = _dot(xb, wv_ref[...], ((1,), (1,)))
    weighted = vals * _dot(ex, sel, ((0,), (0,)))
    r_exp = _dot(r, sel, ((1,), (0,)))
    segvv_ref[...] = segvv_ref[...] * r_exp + _dot(Pt, weighted, ((1,), (0,)))
    m_ref[...] = m_new


def _ln(x, g, b, eps=1e-5):
    mu = jnp.mean(x, axis=-1, keepdims=True)
    var = jnp.mean((x - mu) ** 2, axis=-1, keepdims=True)
    return (x - mu) * lax.rsqrt(var + eps) * g + b


def _final_body(counts_ref, sumx_ref, mparts_ref, d_ref, segvv_ref,
                bv_ref, wo_ref, bo_ref, wg_ref, bg_ref, l1g_ref, l1b_ref,
                wm_ref, bm_ref, wx_ref, bx_ref, ws_ref, bs_ref,
                wgm_ref, bgm_ref, wgx_ref, bgx_ref, wgs_ref, bgs_ref,
                wp_ref, bp_ref, l2g_ref, l2b_ref,
                wf1_ref, bf1_ref, lfg_ref, lfb_ref, wf2_ref, bf2_ref,
                out_ref):
    counts = counts_ref[:, :1]
    nonempty = counts > 0.0
    sum_x = sumx_ref[...]
    avg = sum_x / jnp.maximum(counts, 1.0)

    maxp = mparts_ref[0]
    for t in range(1, NW):
        maxp = jnp.maximum(maxp, mparts_ref[t])
    max_pool = jnp.where(jnp.isfinite(maxp), maxp, 0.0)

    d = d_ref[...]
    d_safe = jnp.where(d == 0.0, 1.0, d)
    wv = segvv_ref[...] / _expand_heads(d_safe)
    wv = wv + bv_ref[...] * nonempty.astype(jnp.float32)
    ctx = _dot(wv, wo_ref[...], ((1,), (1,))) + bo_ref[...]

    comb = jnp.concatenate([ctx, avg], axis=1)
    gate = jax.nn.sigmoid(_dot(comb, wg_ref[...], ((1,), (1,))) + bg_ref[...])
    ctx = gate * ctx + (1.0 - gate) * avg
    ctx = jnp.where(nonempty, ctx, 0.0)
    attn_emb = _ln(ctx, l1g_ref[...], l1b_ref[...])

    mean_r = _dot(avg, wm_ref[...], ((1,), (1,))) + bm_ref[...]
    max_r = _dot(max_pool, wx_ref[...], ((1,), (1,))) + bx_ref[...]
    sum_r = _dot(sum_x, ws_ref[...], ((1,), (1,))) + bs_ref[...]
    gm = jax.nn.sigmoid(
        jnp.sum(mean_r * wgm_ref[...], axis=1, keepdims=True) + bgm_ref[0, 0])
    gx = jax.nn.sigmoid(
        jnp.sum(max_r * wgx_ref[...], axis=1, keepdims=True) + bgx_ref[0, 0])
    gs = jax.nn.sigmoid(
        jnp.sum(sum_r * wgs_ref[...], axis=1, keepdims=True) + bgs_ref[0, 0])
    g0 = jnp.maximum(jnp.maximum(gm, gx), gs)
    em = jnp.exp(gm - g0)
    ex_ = jnp.exp(gx - g0)
    es = jnp.exp(gs - g0)
    z = em + ex_ + es
    pooled = (em * mean_r + ex_ * max_r + es * sum_r) / z
    pool_emb = _ln(_dot(pooled, wp_ref[...], ((1,), (1,))) + bp_ref[...],
                   l2g_ref[...], l2b_ref[...])

    comb2 = jnp.concatenate([attn_emb, pool_emb], axis=1)
    h1 = _dot(comb2, wf1_ref[...], ((1,), (1,))) + bf1_ref[...]
    h1 = _ln(h1, lfg_ref[...], lfb_ref[...])
    h1 = 0.5 * h1 * (1.0 + lax.erf(h1 * (1.0 / math.sqrt(2.0))))
    out_ref[...] = _dot(h1, wf2_ref[...], ((1,), (1,))) + bf2_ref[...]


def _row(v):
    return v.reshape(1, -1)


@jax.jit
def kernel(x, params, batch):
    p = params
    idr = batch.reshape(NBLK, 1, BN)

    maxp_parts = _sc_maxpool(x, batch)

    f32 = jnp.float32
    block = lambda shape: pl.BlockSpec(shape, lambda i: (0,) * len(shape))
    stats = pl.pallas_call(
        _main_body,
        grid=(NBLK,),
        in_specs=[
            pl.BlockSpec((BN, IN), lambda i: (i, 0)),
            pl.BlockSpec((1, 1, BN), lambda i: (i, 0, 0)),
            block((HID, IN)), block((H, DH)), block((HID, 1)),
            block((HID, IN)),
        ],
        out_specs=[
            block((B, 8)), block((B, IN)),
            block((B, H)), block((B, H)), block((B, HID)),
        ],
        out_shape=[
            jax.ShapeDtypeStruct((B, 8), f32),
            jax.ShapeDtypeStruct((B, IN), f32),
            jax.ShapeDtypeStruct((B, H), f32),
            jax.ShapeDtypeStruct((B, H), f32),
            jax.ShapeDtypeStruct((B, HID), f32),
        ],
        scratch_shapes=[
            pltpu.VMEM((H, IN), f32),
            pltpu.VMEM((H, 1), f32),
        ],
    )(x, idr, p['Wk'], p['query'], p['bk'].reshape(HID, 1), p['Wv'])
    counts, sum_x, m_run, d_run, segvv = stats

    out = pl.pallas_call(
        _final_body,
        out_shape=jax.ShapeDtypeStruct((B, IN), f32),
    )(counts, sum_x, maxp_parts, d_run, segvv,
      _row(p['bv']), p['Wo'], _row(p['bo']), p['Wg'], _row(p['bg']),
      _row(p['ln1_g']), _row(p['ln1_b']),
      p['Wm'], _row(p['bm']), p['Wx'], _row(p['bx']), p['Ws'], _row(p['bs']),
      p['Wgm'], _row(p['bgm']), p['Wgx'], _row(p['bgx']),
      p['Wgs'], _row(p['bgs']),
      p['Wp'], _row(p['bp']), _row(p['ln2_g']), _row(p['ln2_b']),
      p['Wf1'], _row(p['bf1']), _row(p['lnf_g']), _row(p['lnf_b']),
      p['Wf2'], _row(p['bf2']))
    return out
